# baseline (speedup 1.0000x reference)
import functools

import jax
import jax.numpy as jnp
from jax import lax
from jax.experimental import pallas as pl
from jax.experimental.pallas import tpu as pltpu
from jax.experimental.pallas import tpu_sc as plsc

NS, L = 16, 16


def _body(B, P, F, PPW,
          perms_hbm, buf_hbm, scores_hbm, new_hbm, idx_out_hbm, out_hbm,
          cand_f, rows_v, scores_v, idxrow_v, idx_all_v, ns_v, bs_v,
          shared_idx, sem_rows, sem_misc, sem_cand):
    wid = lax.axis_index("s")
    p0 = wid * PPW
    rows = B // NS
    base = wid * rows

    cp_rows0 = pltpu.async_copy(buf_hbm.at[0], rows_v, sem_rows)
    cp_cand = pltpu.async_copy(perms_hbm.at[pl.ds(p0, PPW)], cand_f, sem_cand)
    cp_ns = pltpu.async_copy(new_hbm, ns_v, sem_misc)
    cp_bs = pltpu.async_copy(scores_hbm.at[pl.ds(base, rows)], bs_v, sem_misc)
    cp_sc = pltpu.async_copy(scores_hbm, scores_v, sem_misc)

    cp_cand.wait()

    lanes = lax.iota(jnp.int32, L)

    def lane_or(x):
        for sh in (8, 4, 2, 1):
            x = x | x[lanes ^ sh]
        return x

    def all_found(iv):
        pending = jnp.where((iv < 0) & (lanes < PPW), 1, 0)
        return lane_or(pending)[0] == 0

    idxrow_v[...] = jnp.full((L,), -1, jnp.int32)

    def scan_row(b):
        KU = 2

        def kbody(k, accs):
            accs = list(accs)
            for u in range(KU):
                r2 = k * KU + u
                for col in range(0, F, L):
                    r = rows_v[r2, pl.ds(col, L)].astype(jnp.int32)
                    for c in range(PPW):
                        accs[c] = accs[c] | (
                            r ^ cand_f[c, r2, pl.ds(col, L)].astype(jnp.int32))
            return tuple(accs)

        accs = lax.fori_loop(
            0, F // KU, kbody,
            tuple(jnp.zeros((L,), jnp.int32) for _ in range(PPW)))
        iv2 = idxrow_v[...]
        for c in range(PPW):
            match_vec = lane_or(accs[c]) == 0
            iv2 = jnp.where((lanes == c) & (iv2 < 0) & match_vec, b, iv2)
        idxrow_v[...] = iv2

    cp_rows0.wait()
    scan_row(jnp.int32(0))

    @pl.when(jnp.logical_not(all_found(idxrow_v[...])))
    def _():
        def chunk_body(b, dummy):
            @pl.when(jnp.logical_not(all_found(idxrow_v[...])))
            def _():
                pltpu.async_copy(buf_hbm.at[b], rows_v, sem_rows).wait()
                scan_row(b)

            return dummy

        lax.fori_loop(1, B, chunk_body, jnp.int32(0))

    iv = idxrow_v[...]
    cp_sc.wait()

    def gbody(ch, g):
        svec = scores_v[pl.ds(ch * L, L)]
        val = svec[iv & (L - 1)]
        return jnp.where((iv >> 4) == ch, val, g)

    g = lax.fori_loop(0, B // L, gbody, jnp.full((L,), -jnp.inf, jnp.float32))
    iv = jnp.where((iv >= 0) & (g != -jnp.inf), iv, -1)
    idxrow_v[...] = iv
    cp_idx = pltpu.async_copy(
        idxrow_v.at[pl.ds(0, PPW)], idx_out_hbm.at[pl.ds(p0, PPW)], sem_cand)

    pltpu.sync_copy(idxrow_v.at[pl.ds(0, PPW)], shared_idx.at[pl.ds(p0, PPW)])
    plsc.subcore_barrier()
    pltpu.sync_copy(shared_idx, idx_all_v)
    cp_ns.wait()
    cp_bs.wait()

    nvec = rows // L
    lanevs = [base + v * L + lanes for v in range(nvec)]

    def tbody(t, accs):
        nsvec = ns_v[pl.ds(t * L, L)]
        ivec = idx_all_v[pl.ds(t * L, L)]
        for j in range(L):
            ip = ivec[j]
            sp = nsvec[j]
            accs = tuple(
                jnp.where(lanevs[v] == ip, jnp.maximum(accs[v], sp), accs[v])
                for v in range(nvec))
        return accs

    accs = lax.fori_loop(
        0, P // L, tbody, tuple(bs_v[pl.ds(v * L, L)] for v in range(nvec)))
    for v in range(nvec):
        bs_v[pl.ds(v * L, L)] = accs[v]
    pltpu.sync_copy(bs_v, out_hbm.at[pl.ds(base, rows)])
    cp_idx.wait()


def kernel(perm_buffer, buffer_scores, permutations, new_scores):
    B = perm_buffer.shape[0]
    P = permutations.shape[0]
    F = perm_buffer.shape[1]
    PPW = P // NS

    mesh = plsc.VectorSubcoreMesh(
        core_axis_name="c", subcore_axis_name="s", num_cores=1)

    run = functools.partial(
        pl.kernel,
        out_type=(
            jax.ShapeDtypeStruct((P,), jnp.int32),
            jax.ShapeDtypeStruct((B,), jnp.float32),
        ),
        mesh=mesh,
        scratch_types=[
            pltpu.VMEM((PPW, F, F), jnp.float32),
            pltpu.VMEM((F, F), jnp.float32),
            pltpu.VMEM((B,), jnp.float32),
            pltpu.VMEM((L,), jnp.int32),
            pltpu.VMEM((P,), jnp.int32),
            pltpu.VMEM((P,), jnp.float32),
            pltpu.VMEM((B // NS,), jnp.float32),
            pltpu.MemorySpace.VMEM_SHARED((P,), jnp.int32),
            pltpu.SemaphoreType.DMA,
            pltpu.SemaphoreType.DMA,
            pltpu.SemaphoreType.DMA,
        ],
    )(functools.partial(_body, B, P, F, PPW))
    idx, updated = run(permutations, perm_buffer, buffer_scores, new_scores)
    return (idx, updated)


# speedup vs baseline: 1.2946x; 1.2946x over previous
import functools

import jax
import jax.numpy as jnp
from jax import lax
from jax.experimental import pallas as pl
from jax.experimental.pallas import tpu as pltpu
from jax.experimental.pallas import tpu_sc as plsc

NS, L = 16, 16


def _body(B, P, FLAT, PPW, CH,
          perms_hbm, buf_hbm, scores_hbm, new_hbm, idx_out_hbm, out_hbm,
          cand_f, rows_v, scores_v, idxrow_v, idx_all_v, ns_v, bs_v,
          shared_idx, sem_rows, sem_misc, sem_cand):
    wid = lax.axis_index("s")
    p0 = wid * PPW
    rows = B // NS
    base = wid * rows

    cp_rows0 = pltpu.async_copy(buf_hbm.at[pl.ds(0, CH)], rows_v, sem_rows)
    cp_cand = pltpu.async_copy(perms_hbm.at[pl.ds(p0, PPW)], cand_f, sem_cand)
    cp_ns = pltpu.async_copy(new_hbm, ns_v, sem_misc)
    cp_bs = pltpu.async_copy(scores_hbm.at[pl.ds(base, rows)], bs_v, sem_misc)
    cp_sc = pltpu.async_copy(scores_hbm, scores_v, sem_misc)

    cp_cand.wait()
    nk = FLAT // L

    lanes = lax.iota(jnp.int32, L)

    def lane_or(x):
        for sh in (8, 4, 2, 1):
            x = x | x[lanes ^ sh]
        return x

    def all_found(iv):
        pending = jnp.where((iv < 0) & (lanes < PPW), 1, 0)
        return lane_or(pending)[0] == 0

    idxrow_v[...] = jnp.full((L,), -1, jnp.int32)

    def scan_rows(chunk):
        def row_body(rb, d2):
            iv = idxrow_v[...]

            @pl.when(jnp.logical_not(all_found(iv)))
            def _():
                KU = 4

                def kbody(k, accs):
                    accs = list(accs)
                    for u in range(KU):
                        off = (k * KU + u) * L
                        r = rows_v[rb, pl.ds(off, L)].astype(jnp.int32)
                        for c in range(PPW):
                            accs[c] = accs[c] | (
                                r ^ cand_f[c, pl.ds(off, L)].astype(jnp.int32))
                    return tuple(accs)

                accs = lax.fori_loop(
                    0, nk // KU, kbody,
                    tuple(jnp.zeros((L,), jnp.int32) for _ in range(PPW)))
                b = chunk * CH + rb
                iv2 = iv
                for c in range(PPW):
                    match_vec = lane_or(accs[c]) == 0
                    iv2 = jnp.where((lanes == c) & (iv2 < 0) & match_vec, b, iv2)
                idxrow_v[...] = iv2

            return d2

        lax.fori_loop(0, CH, row_body, jnp.int32(0))

    cp_rows0.wait()
    scan_rows(jnp.int32(0))

    @pl.when(jnp.logical_not(all_found(idxrow_v[...])))
    def _():
        def chunk_body(chunk, dummy):
            @pl.when(jnp.logical_not(all_found(idxrow_v[...])))
            def _():
                pltpu.async_copy(
                    buf_hbm.at[pl.ds(chunk * CH, CH)], rows_v, sem_rows).wait()
                scan_rows(chunk)

            return dummy

        lax.fori_loop(1, B // CH, chunk_body, jnp.int32(0))

    iv = idxrow_v[...]
    cp_sc.wait()

    def gbody(ch, g):
        svec = scores_v[pl.ds(ch * L, L)]
        val = svec[iv & (L - 1)]
        return jnp.where((iv >> 4) == ch, val, g)

    g = lax.fori_loop(0, B // L, gbody, jnp.full((L,), -jnp.inf, jnp.float32))
    iv = jnp.where((iv >= 0) & (g != -jnp.inf), iv, -1)
    idxrow_v[...] = iv
    cp_idx = pltpu.async_copy(
        idxrow_v.at[pl.ds(0, PPW)], idx_out_hbm.at[pl.ds(p0, PPW)], sem_cand)

    pltpu.sync_copy(idxrow_v.at[pl.ds(0, PPW)], shared_idx.at[pl.ds(p0, PPW)])
    plsc.subcore_barrier()
    pltpu.sync_copy(shared_idx, idx_all_v)
    cp_ns.wait()
    cp_bs.wait()

    nvec = rows // L
    lanevs = [base + v * L + lanes for v in range(nvec)]

    def tbody(t, accs):
        nsvec = ns_v[pl.ds(t * L, L)]
        ivec = idx_all_v[pl.ds(t * L, L)]
        for j in range(L):
            ip = ivec[j]
            sp = nsvec[j]
            accs = tuple(
                jnp.where(lanevs[v] == ip, jnp.maximum(accs[v], sp), accs[v])
                for v in range(nvec))
        return accs

    accs = lax.fori_loop(
        0, P // L, tbody, tuple(bs_v[pl.ds(v * L, L)] for v in range(nvec)))
    for v in range(nvec):
        bs_v[pl.ds(v * L, L)] = accs[v]
    pltpu.sync_copy(bs_v, out_hbm.at[pl.ds(base, rows)])
    cp_idx.wait()


def kernel(perm_buffer, buffer_scores, permutations, new_scores):
    B = perm_buffer.shape[0]
    P = permutations.shape[0]
    FLAT = perm_buffer.shape[1] * perm_buffer.shape[2]
    PPW = P // NS
    CH = 1

    buf2d = perm_buffer.reshape(B, FLAT)
    perm2d = permutations.reshape(P, FLAT)

    mesh = plsc.VectorSubcoreMesh(
        core_axis_name="c", subcore_axis_name="s", num_cores=1)

    run = functools.partial(
        pl.kernel,
        out_type=(
            jax.ShapeDtypeStruct((P,), jnp.int32),
            jax.ShapeDtypeStruct((B,), jnp.float32),
        ),
        mesh=mesh,
        scratch_types=[
            pltpu.VMEM((PPW, FLAT), jnp.float32),
            pltpu.VMEM((CH, FLAT), jnp.float32),
            pltpu.VMEM((B,), jnp.float32),
            pltpu.VMEM((L,), jnp.int32),
            pltpu.VMEM((P,), jnp.int32),
            pltpu.VMEM((P,), jnp.float32),
            pltpu.VMEM((B // NS,), jnp.float32),
            pltpu.MemorySpace.VMEM_SHARED((P,), jnp.int32),
            pltpu.SemaphoreType.DMA,
            pltpu.SemaphoreType.DMA,
            pltpu.SemaphoreType.DMA,
        ],
    )(functools.partial(_body, B, P, FLAT, PPW, CH))
    idx, updated = run(perm2d, buf2d, buffer_scores, new_scores)
    return (idx, updated)
